# Initial kernel scaffold; baseline (speedup 1.0000x reference)
import functools

import jax
import jax.numpy as jnp
from jax import lax
from jax.experimental import pallas as pl
from jax.experimental.pallas import tpu as pltpu
from jax.experimental.pallas import tpu_sc as plsc

NUM_USERS = 2500
NUM_ITEMS = 7500
N_NODES = NUM_USERS + NUM_ITEMS
D = 128
E_A = 320000
E_S = 80000

CHUNK = 128
NC, NS = 2, 16
NW = NC * NS
U_PAD = 2560
A_CHUNKS = E_A // CHUNK
S_CHUNKS = E_S // CHUNK
A_ROWS_PER_TILE = N_NODES // NS
S_ROWS_PER_TILE = U_PAD // NS


def _spmm_pair_body(ea_src, ea_dst, a_vals, es_src, es_dst, s_vals,
                    x_ego, x_soc, zeros_hbm,
                    out_a, out_s,
                    acc_a, acc_s, idx_s, idx_d, vals_v, rows, sem):
    cid = lax.axis_index("c")
    sid = lax.axis_index("s")
    wid = sid * NC + cid

    pltpu.sync_copy(zeros_hbm.at[pl.ds(sid * A_ROWS_PER_TILE, A_ROWS_PER_TILE)],
                    acc_a.at[pl.ds(sid * A_ROWS_PER_TILE, A_ROWS_PER_TILE)])
    pltpu.sync_copy(zeros_hbm.at[pl.ds(sid * S_ROWS_PER_TILE, S_ROWS_PER_TILE)],
                    acc_s.at[pl.ds(sid * S_ROWS_PER_TILE, S_ROWS_PER_TILE)])
    plsc.subcore_barrier()

    def make_chunk(src_hbm, dst_hbm, vals_hbm, x_hbm, acc):
        def chunk(k, carry):
            base = (wid + k * NW) * CHUNK
            pltpu.sync_copy(src_hbm.at[pl.ds(base, CHUNK)], idx_s)
            pltpu.sync_copy(dst_hbm.at[pl.ds(base, CHUNK)], idx_d)
            pltpu.sync_copy(vals_hbm.at[pl.ds(base, CHUNK)], vals_v)
            pltpu.async_copy(x_hbm.at[idx_s], rows, sem).wait()

            def row_scale(j, c2):
                v = vals_v[j]
                for h in range(D // 16):
                    sl = pl.ds(h * 16, 16)
                    rows[j, sl] = rows[j, sl] * v
                return c2
            lax.fori_loop(0, CHUNK, row_scale, 0)
            pltpu.sync_copy(rows, acc.at[idx_d], add=True)
            return carry
        return chunk

    n_a = (A_CHUNKS - wid + NW - 1) // NW
    lax.fori_loop(0, n_a, make_chunk(ea_src, ea_dst, a_vals, x_ego, acc_a), 0)
    n_s = (S_CHUNKS - wid + NW - 1) // NW
    lax.fori_loop(0, n_s, make_chunk(es_src, es_dst, s_vals, x_soc, acc_s), 0)

    plsc.subcore_barrier()
    pltpu.sync_copy(acc_a.at[pl.ds(sid * A_ROWS_PER_TILE, A_ROWS_PER_TILE)],
                    out_a.at[cid, pl.ds(sid * A_ROWS_PER_TILE, A_ROWS_PER_TILE)])
    pltpu.sync_copy(acc_s.at[pl.ds(sid * S_ROWS_PER_TILE, S_ROWS_PER_TILE)],
                    out_s.at[cid, pl.ds(sid * S_ROWS_PER_TILE, S_ROWS_PER_TILE)])


_spmm_pair = pl.kernel(
    _spmm_pair_body,
    out_type=(jax.ShapeDtypeStruct((NC, N_NODES, D), jnp.float32),
              jax.ShapeDtypeStruct((NC, U_PAD, D), jnp.float32)),
    mesh=plsc.VectorSubcoreMesh(core_axis_name="c", subcore_axis_name="s"),
    scratch_types=[
        pltpu.VMEM_SHARED((N_NODES, D), jnp.float32),
        pltpu.VMEM_SHARED((U_PAD, D), jnp.float32),
        pltpu.VMEM((CHUNK,), jnp.int32),
        pltpu.VMEM((CHUNK,), jnp.int32),
        pltpu.VMEM((CHUNK,), jnp.float32),
        pltpu.VMEM((CHUNK, D), jnp.float32),
        pltpu.SemaphoreType.DMA,
    ],
)


def _mish(x):
    return x * jnp.tanh(jax.nn.softplus(x))


def _fusion_core(ep_ref, sp_ref, w1, b1, w2, b2, w3, b3):
    u = ep_ref[0, :NUM_USERS, :] + ep_ref[1, :NUM_USERS, :]
    s = sp_ref[0, :NUM_USERS, :] + sp_ref[1, :NUM_USERS, :]
    c = jnp.concatenate([u, s, u * s], axis=1)
    t1 = _mish(jnp.dot(c, w1[...], preferred_element_type=jnp.float32) + b1[...])
    t2 = _mish(jnp.dot(t1, w2[...], preferred_element_type=jnp.float32) + b2[...])
    t3 = jnp.dot(t2, w3[...], preferred_element_type=jnp.float32) + b3[...]
    soc = t3 / jnp.sqrt(jnp.sum(t3 * t3))
    items = ep_ref[0, NUM_USERS:, :] + ep_ref[1, NUM_USERS:, :]
    return soc, items


def _fusion_mid_body(ep_ref, sp_ref, w1, b1, w2, b2, w3, b3, ego_out, soc_out):
    soc, items = _fusion_core(ep_ref, sp_ref, w1, b1, w2, b2, w3, b3)
    ego_out[:NUM_USERS, :] = soc
    ego_out[NUM_USERS:, :] = items
    soc_out[...] = soc


def _fusion_final_body(ep_ref, sp_ref, w1, b1, w2, b2, w3, b3,
                       ego0_ref, ego1_ref, u_out, i_out):
    soc, items = _fusion_core(ep_ref, sp_ref, w1, b1, w2, b2, w3, b3)
    u_out[...] = (ego0_ref[:NUM_USERS, :] + ego1_ref[:NUM_USERS, :] + soc) / 3.0
    i_out[...] = (ego0_ref[NUM_USERS:, :] + ego1_ref[NUM_USERS:, :] + items) / 3.0


_fusion_mid = pl.pallas_call(
    _fusion_mid_body,
    out_shape=(jax.ShapeDtypeStruct((N_NODES, D), jnp.float32),
               jax.ShapeDtypeStruct((NUM_USERS, D), jnp.float32)),
)

_fusion_final = pl.pallas_call(
    _fusion_final_body,
    out_shape=(jax.ShapeDtypeStruct((NUM_USERS, D), jnp.float32),
               jax.ShapeDtypeStruct((NUM_ITEMS, D), jnp.float32)),
)


def kernel(user_emb, item_emb, a_vals, s_vals, fc1_w, fc1_b, fc2_w, fc2_b,
           fc3_w, fc3_b, edge_index_a, edge_index_s):
    ego0 = jnp.concatenate([user_emb, item_emb], axis=0)
    soc0 = user_emb
    zeros = jnp.zeros((N_NODES, D), jnp.float32)
    ea_src, ea_dst = edge_index_a[0], edge_index_a[1]
    es_src, es_dst = edge_index_s[0], edge_index_s[1]

    ep1, sp1 = _spmm_pair(ea_src, ea_dst, a_vals, es_src, es_dst, s_vals,
                          ego0, soc0, zeros)
    ego1, soc1 = _fusion_mid(ep1, sp1, fc1_w, fc1_b, fc2_w, fc2_b, fc3_w, fc3_b)
    ep2, sp2 = _spmm_pair(ea_src, ea_dst, a_vals, es_src, es_dst, s_vals,
                          ego1, soc1, zeros)
    u_mean, i_mean = _fusion_final(ep2, sp2, fc1_w, fc1_b, fc2_w, fc2_b,
                                   fc3_w, fc3_b, ego0, ego1)
    return u_mean, i_mean


# speedup vs baseline: 4.1592x; 4.1592x over previous
import functools

import jax
import jax.numpy as jnp
from jax import lax
from jax.experimental import pallas as pl
from jax.experimental.pallas import tpu as pltpu
from jax.experimental.pallas import tpu_sc as plsc

NUM_USERS = 2500
NUM_ITEMS = 7500
N_NODES = NUM_USERS + NUM_ITEMS
D = 128
E_A = 320000
E_S = 80000

CHUNK = 128
NC, NS = 2, 16
NW = NC * NS
U_PAD = 2560
A_CHUNKS = E_A // CHUNK
S_CHUNKS = E_S // CHUNK
N_PAD = 10240
A_ROWS_PER_TILE = N_PAD // NS
S_ROWS_PER_TILE = U_PAD // NS


def _spmm_pair_body(ea_src, ea_dst, a_vals, es_src, es_dst, s_vals,
                    x_ego, x_soc, zeros_hbm,
                    out_a, out_s,
                    acc_a, acc_s, idx_s, idx_d, vals_v, rows, sem):
    cid = lax.axis_index("c")
    sid = lax.axis_index("s")
    wid = sid * NC + cid

    pltpu.sync_copy(zeros_hbm.at[pl.ds(sid * A_ROWS_PER_TILE, A_ROWS_PER_TILE)],
                    acc_a.at[pl.ds(sid * A_ROWS_PER_TILE, A_ROWS_PER_TILE)])
    pltpu.sync_copy(zeros_hbm.at[pl.ds(sid * S_ROWS_PER_TILE, S_ROWS_PER_TILE)],
                    acc_s.at[pl.ds(sid * S_ROWS_PER_TILE, S_ROWS_PER_TILE)])
    plsc.subcore_barrier()

    def make_chunk(src_hbm, dst_hbm, vals_hbm, x_hbm, acc):
        def chunk(k, carry):
            base = (wid + k * NW) * CHUNK
            pltpu.sync_copy(src_hbm.at[pl.ds(base, CHUNK)], idx_s)
            pltpu.sync_copy(dst_hbm.at[pl.ds(base, CHUNK)], idx_d)
            pltpu.sync_copy(vals_hbm.at[pl.ds(base, CHUNK)], vals_v)
            pltpu.async_copy(x_hbm.at[idx_s], rows, sem).wait()

            def group_scale(g, c2):
                vv = vals_v[pl.ds(g * 16, 16)]
                for i in range(16):
                    bc = vv.at[jnp.full((16,), i, jnp.int32)].get(
                        mode="promise_in_bounds")
                    r = g * 16 + i
                    for h in range(D // 16):
                        sl = pl.ds(h * 16, 16)
                        rows[r, sl] = rows[r, sl] * bc
                return c2
            lax.fori_loop(0, CHUNK // 16, group_scale, 0)
            pltpu.sync_copy(rows, acc.at[idx_d], add=True)
            return carry
        return chunk

    n_a = (A_CHUNKS - wid + NW - 1) // NW
    lax.fori_loop(0, n_a, make_chunk(ea_src, ea_dst, a_vals, x_ego, acc_a), 0)
    n_s = (S_CHUNKS - wid + NW - 1) // NW
    lax.fori_loop(0, n_s, make_chunk(es_src, es_dst, s_vals, x_soc, acc_s), 0)

    plsc.subcore_barrier()
    pltpu.sync_copy(acc_a.at[pl.ds(sid * A_ROWS_PER_TILE, A_ROWS_PER_TILE)],
                    out_a.at[cid, pl.ds(sid * A_ROWS_PER_TILE, A_ROWS_PER_TILE)])
    pltpu.sync_copy(acc_s.at[pl.ds(sid * S_ROWS_PER_TILE, S_ROWS_PER_TILE)],
                    out_s.at[cid, pl.ds(sid * S_ROWS_PER_TILE, S_ROWS_PER_TILE)])


_spmm_pair = pl.kernel(
    _spmm_pair_body,
    out_type=(jax.ShapeDtypeStruct((NC, N_PAD, D), jnp.float32),
              jax.ShapeDtypeStruct((NC, U_PAD, D), jnp.float32)),
    mesh=plsc.VectorSubcoreMesh(core_axis_name="c", subcore_axis_name="s"),
    scratch_types=[
        pltpu.VMEM_SHARED((N_PAD, D), jnp.float32),
        pltpu.VMEM_SHARED((U_PAD, D), jnp.float32),
        pltpu.VMEM((CHUNK,), jnp.int32),
        pltpu.VMEM((CHUNK,), jnp.int32),
        pltpu.VMEM((CHUNK,), jnp.float32),
        pltpu.VMEM((CHUNK, D), jnp.float32),
        pltpu.SemaphoreType.DMA,
    ],
)


def _mish(x):
    return x * jnp.tanh(jax.nn.softplus(x))


def _fusion_core(ep_ref, sp_ref, w1, b1, w2, b2, w3, b3):
    u = ep_ref[0, :NUM_USERS, :] + ep_ref[1, :NUM_USERS, :]
    s = sp_ref[0, :NUM_USERS, :] + sp_ref[1, :NUM_USERS, :]
    c = jnp.concatenate([u, s, u * s], axis=1)
    t1 = _mish(jnp.dot(c, w1[...], preferred_element_type=jnp.float32) + b1[...])
    t2 = _mish(jnp.dot(t1, w2[...], preferred_element_type=jnp.float32) + b2[...])
    t3 = jnp.dot(t2, w3[...], preferred_element_type=jnp.float32) + b3[...]
    soc = t3 / jnp.sqrt(jnp.sum(t3 * t3))
    items = ep_ref[0, NUM_USERS:N_NODES, :] + ep_ref[1, NUM_USERS:N_NODES, :]
    return soc, items


def _fusion_mid_body(ep_ref, sp_ref, w1, b1, w2, b2, w3, b3, ego_out, soc_out):
    soc, items = _fusion_core(ep_ref, sp_ref, w1, b1, w2, b2, w3, b3)
    ego_out[:NUM_USERS, :] = soc
    ego_out[NUM_USERS:, :] = items
    soc_out[...] = soc


def _fusion_final_body(ep_ref, sp_ref, w1, b1, w2, b2, w3, b3,
                       ego0_ref, ego1_ref, u_out, i_out):
    soc, items = _fusion_core(ep_ref, sp_ref, w1, b1, w2, b2, w3, b3)
    u_out[...] = (ego0_ref[:NUM_USERS, :] + ego1_ref[:NUM_USERS, :] + soc) / 3.0
    i_out[...] = (ego0_ref[NUM_USERS:, :] + ego1_ref[NUM_USERS:, :] + items) / 3.0


_fusion_mid = pl.pallas_call(
    _fusion_mid_body,
    out_shape=(jax.ShapeDtypeStruct((N_NODES, D), jnp.float32),
               jax.ShapeDtypeStruct((NUM_USERS, D), jnp.float32)),
)

_fusion_final = pl.pallas_call(
    _fusion_final_body,
    out_shape=(jax.ShapeDtypeStruct((NUM_USERS, D), jnp.float32),
               jax.ShapeDtypeStruct((NUM_ITEMS, D), jnp.float32)),
)


def kernel(user_emb, item_emb, a_vals, s_vals, fc1_w, fc1_b, fc2_w, fc2_b,
           fc3_w, fc3_b, edge_index_a, edge_index_s):
    ego0 = jnp.concatenate([user_emb, item_emb], axis=0)
    soc0 = user_emb
    zeros = jnp.zeros((N_PAD, D), jnp.float32)
    ea_src, ea_dst = edge_index_a[0], edge_index_a[1]
    es_src, es_dst = edge_index_s[0], edge_index_s[1]

    ep1, sp1 = _spmm_pair(ea_src, ea_dst, a_vals, es_src, es_dst, s_vals,
                          ego0, soc0, zeros)
    ego1, soc1 = _fusion_mid(ep1, sp1, fc1_w, fc1_b, fc2_w, fc2_b, fc3_w, fc3_b)
    ep2, sp2 = _spmm_pair(ea_src, ea_dst, a_vals, es_src, es_dst, s_vals,
                          ego1, soc1, zeros)
    u_mean, i_mean = _fusion_final(ep2, sp2, fc1_w, fc1_b, fc2_w, fc2_b,
                                   fc3_w, fc3_b, ego0, ego1)
    return u_mean, i_mean
